# baseline (speedup 1.0000x reference)
import jax
import jax.numpy as jnp
from jax import lax
from jax.experimental import pallas as pl
from jax.experimental.pallas import tpu as pltpu
from jax.experimental.pallas import tpu_sc as plsc

VOCAB = 100000
N_EMBD = 128
N_CTX = 2048
BATCH = 64
SEQ = 2048

NC = 2
NS = 16
NW = NC * NS
LANES = 16

ROWS_PER_C = 128
NCHUNK = BATCH * SEQ // (NW * ROWS_PER_C)
NBUF = 3
POS_STAGE = SEQ // NS


def _embed_body(ids_hbm, tok_hbm, pos_hbm, out_hbm,
                idx_all, iota_v, rows0, rows1, rows2, pos_sp,
                gs0, gs1, gs2, ps0, ps1, ps2, ws0, ws1, ws2):
    cid = lax.axis_index("c")
    sid = lax.axis_index("s")
    wid = sid * NC + cid
    rows = (rows0, rows1, rows2)
    gsem = (gs0, gs1, gs2)
    psem = (ps0, ps1, ps2)
    wsem = (ws0, ws1, ws2)

    pltpu.sync_copy(pos_hbm.at[pl.ds(sid * POS_STAGE, POS_STAGE)], rows0)
    pltpu.sync_copy(rows0, pos_sp.at[pl.ds(sid * POS_STAGE, POS_STAGE)])
    pltpu.sync_copy(ids_hbm.at[wid], idx_all)

    def iota_body(t, _):
        iota_v[pl.ds(t * LANES, LANES)] = lax.iota(jnp.int32, LANES) + t * LANES
        return 0

    lax.fori_loop(0, SEQ // LANES, iota_body, 0)
    plsc.subcore_barrier()

    def fire_gather(c, k):
        pltpu.make_async_copy(tok_hbm.at[idx_all.at[c]], rows[k], gsem[k]).start()

    def wait_gather(c, k):
        pltpu.make_async_copy(tok_hbm.at[idx_all.at[c]], rows[k], gsem[k]).wait()

    def posidx(c):
        return iota_v.at[pl.ds((c % (SEQ // ROWS_PER_C)) * ROWS_PER_C,
                               ROWS_PER_C)]

    def fire_posadd(c, k):
        pltpu.make_async_copy(pos_sp.at[posidx(c)], rows[k],
                              psem[k]).start(add=True)

    def wait_posadd(c, k):
        pltpu.make_async_copy(pos_sp.at[posidx(c)], rows[k], psem[k]).wait()

    def fire_write(c, k):
        pltpu.make_async_copy(rows[k], out_hbm.at[wid, c], wsem[k]).start()

    def wait_write(c, k):
        pltpu.make_async_copy(rows[k], out_hbm.at[wid, c], wsem[k]).wait()

    fire_gather(0, 0)

    fire_gather(1, 1)
    wait_gather(0, 0)
    fire_posadd(0, 0)

    fire_gather(2, 2)
    wait_gather(1, 1)
    fire_posadd(1, 1)
    wait_posadd(0, 0)
    fire_write(0, 0)

    wait_write(0, 0)
    fire_gather(3, 0)
    wait_gather(2, 2)
    fire_posadd(2, 2)
    wait_posadd(1, 1)
    fire_write(1, 1)

    def group(g, _):
        for u in range(NBUF):
            c = g * NBUF + u
            k = u
            kn = (u + 1) % NBUF
            kp = (u + 2) % NBUF
            wait_write(c - 2, kn)
            fire_gather(c + 1, kn)
            wait_gather(c, k)
            fire_posadd(c, k)
            wait_posadd(c - 1, kp)
            fire_write(c - 1, kp)
        return 0

    lax.fori_loop(1, (NCHUNK - 2) // NBUF, group, 0)

    c = NCHUNK - 2
    wait_write(c - 2, 1)
    fire_gather(c + 1, 1)
    wait_gather(c, 0)
    fire_posadd(c, 0)
    wait_posadd(c - 1, 2)
    fire_write(c - 1, 2)

    c = NCHUNK - 1
    wait_write(c - 2, 2)
    wait_gather(c, 1)
    fire_posadd(c, 1)
    wait_posadd(c - 1, 0)
    fire_write(c - 1, 0)
    wait_posadd(c, 1)
    fire_write(c, 1)
    wait_write(c - 1, 0)
    wait_write(c, 1)


@jax.jit
def _embed(ids_r, tok_table, pos_table):
    mesh = plsc.VectorSubcoreMesh(core_axis_name="c", subcore_axis_name="s")
    return pl.kernel(
        _embed_body,
        out_type=jax.ShapeDtypeStruct((NW, NCHUNK, ROWS_PER_C, N_EMBD),
                                      jnp.float32),
        mesh=mesh,
        scratch_types=[
            pltpu.VMEM((NCHUNK, ROWS_PER_C), jnp.int32),
            pltpu.VMEM((SEQ,), jnp.int32),
            pltpu.VMEM((ROWS_PER_C, N_EMBD), jnp.float32),
            pltpu.VMEM((ROWS_PER_C, N_EMBD), jnp.float32),
            pltpu.VMEM((ROWS_PER_C, N_EMBD), jnp.float32),
            pltpu.VMEM_SHARED((SEQ, N_EMBD), jnp.float32),
            pltpu.SemaphoreType.DMA,
            pltpu.SemaphoreType.DMA,
            pltpu.SemaphoreType.DMA,
            pltpu.SemaphoreType.DMA,
            pltpu.SemaphoreType.DMA,
            pltpu.SemaphoreType.DMA,
            pltpu.SemaphoreType.DMA,
            pltpu.SemaphoreType.DMA,
            pltpu.SemaphoreType.DMA,
        ],
    )(ids_r, tok_table, pos_table)


def kernel(ids, tok_table, pos_table):
    B, L = ids.shape
    ids_r = ids.astype(jnp.int32).reshape(NW, NCHUNK, ROWS_PER_C)
    out = _embed(ids_r, tok_table, pos_table)
    attn_mask = jnp.ones((B, 1, 1, L), dtype=bool)
    return out.reshape(B, L, N_EMBD), attn_mask
